# Initial kernel scaffold; baseline (speedup 1.0000x reference)
import jax
import jax.numpy as jnp
from jax.experimental import pallas as pl


def kernel(token_ids, weights):
    raise NotImplementedError("write your pallas kernel here")


# speedup vs baseline: 1.8431x; 1.8431x over previous
import functools

import jax
import jax.numpy as jnp
from jax import lax
from jax.experimental import pallas as pl
from jax.experimental.pallas import tpu as pltpu
from jax.experimental.pallas import tpu_sc as plsc

NUM_EMBEDDINGS = 1000000
EMBEDDING_DIM = 64
BATCH = 16384
HIST = 50

_N = BATCH * HIST
_IDX_MINOR = 128
_K = 8
_CHUNK = _K * _IDX_MINOR

_info = plsc.get_sparse_core_info()
_NC = _info.num_cores
_NS = _info.num_subcores
_NW = _NC * _NS
_PER_W = _N // _NW
_ROWS_PER_W = _PER_W // _IDX_MINOR
_STEPS = _ROWS_PER_W // _K


def _sc_gather(idx2d, weights):
    mesh = plsc.VectorSubcoreMesh(core_axis_name="c", subcore_axis_name="s")

    @functools.partial(
        pl.kernel,
        mesh=mesh,
        out_type=jax.ShapeDtypeStruct((_N, EMBEDDING_DIM), jnp.float32),
        scratch_types=[
            pltpu.VMEM((_K, _IDX_MINOR), jnp.int32),
            pltpu.VMEM((_CHUNK, EMBEDDING_DIM), jnp.float32),
            pltpu.SemaphoreType.DMA,
        ],
        compiler_params=pltpu.CompilerParams(use_tc_tiling_on_sc=False),
    )
    def body(idx_hbm, table_hbm, out_hbm, idx_v, rows_v, sem):
        wid = lax.axis_index("s") * _NC + lax.axis_index("c")
        row0 = wid * _ROWS_PER_W

        def step(g, carry):
            r = row0 + g * _K
            pltpu.sync_copy(idx_hbm.at[pl.ds(r, _K)], idx_v)
            copies = []
            for j in range(_K):
                copies.append(
                    pltpu.async_copy(
                        table_hbm.at[idx_v.at[j]],
                        rows_v.at[pl.ds(j * _IDX_MINOR, _IDX_MINOR)],
                        sem,
                    )
                )
            for c in copies:
                c.wait()
            pltpu.sync_copy(rows_v, out_hbm.at[pl.ds(r * _IDX_MINOR, _CHUNK)])
            return carry

        lax.fori_loop(0, _STEPS, step, 0)

    return body(idx2d, weights)


def kernel(token_ids, weights):
    idx2d = token_ids.astype(jnp.int32).reshape(_N // _IDX_MINOR, _IDX_MINOR)
    out = _sc_gather(idx2d, weights)
    return out.reshape(BATCH, HIST, EMBEDDING_DIM)


# speedup vs baseline: 1.8730x; 1.0162x over previous
import functools

import jax
import jax.numpy as jnp
from jax import lax
from jax.experimental import pallas as pl
from jax.experimental.pallas import tpu as pltpu
from jax.experimental.pallas import tpu_sc as plsc

NUM_EMBEDDINGS = 1000000
EMBEDDING_DIM = 64
BATCH = 16384
HIST = 50

_N = BATCH * HIST
_IDX_MINOR = 128
_K = 5
_CHUNK = _K * _IDX_MINOR

_info = plsc.get_sparse_core_info()
_NC = _info.num_cores
_NS = _info.num_subcores
_NW = _NC * _NS
_PER_W = _N // _NW
_ROWS_PER_W = _PER_W // _IDX_MINOR
_NCHUNKS = _ROWS_PER_W // _K
_PAIRS = _NCHUNKS // 2


def _sc_gather(idx2d, weights):
    mesh = plsc.VectorSubcoreMesh(core_axis_name="c", subcore_axis_name="s")

    @functools.partial(
        pl.kernel,
        mesh=mesh,
        out_type=jax.ShapeDtypeStruct((_N, EMBEDDING_DIM), jnp.float32),
        scratch_types=[
            pltpu.VMEM((_K, _IDX_MINOR), jnp.int32),
            pltpu.VMEM((_K, _IDX_MINOR), jnp.int32),
            pltpu.VMEM((_CHUNK, EMBEDDING_DIM), jnp.float32),
            pltpu.VMEM((_CHUNK, EMBEDDING_DIM), jnp.float32),
            pltpu.SemaphoreType.DMA,
            pltpu.SemaphoreType.DMA,
            pltpu.SemaphoreType.DMA,
            pltpu.SemaphoreType.DMA,
            pltpu.SemaphoreType.DMA,
            pltpu.SemaphoreType.DMA,
        ],
        compiler_params=pltpu.CompilerParams(use_tc_tiling_on_sc=False),
    )
    def body(idx_hbm, table_hbm, out_hbm, idx0, idx1, rows0, rows1,
             isem0, isem1, gsem0, gsem1, osem0, osem1):
        wid = lax.axis_index("s") * _NC + lax.axis_index("c")
        row0 = wid * _ROWS_PER_W
        slots = ((idx0, rows0, isem0, gsem0, osem0),
                 (idx1, rows1, isem1, gsem1, osem1))

        pltpu.async_copy(idx_hbm.at[pl.ds(row0, _K)], idx0, isem0)
        pltpu.async_copy(idx_hbm.at[pl.ds(row0 + _K, _K)], idx1, isem1)

        def step(p, carry):
            for b, (idxv, rowsv, isem, gsem, osem) in enumerate(slots):
                r = row0 + (2 * p + b) * _K
                pltpu.make_async_copy(
                    idx_hbm.at[pl.ds(r, _K)], idxv, isem).wait()

                @pl.when(p > 0)
                def _():
                    pltpu.make_async_copy(
                        rowsv,
                        out_hbm.at[pl.ds((r - 2 * _K) * _IDX_MINOR, _CHUNK)],
                        osem).wait()

                gathers = [
                    pltpu.async_copy(
                        table_hbm.at[idxv.at[j]],
                        rowsv.at[pl.ds(j * _IDX_MINOR, _IDX_MINOR)],
                        gsem,
                    )
                    for j in range(_K)
                ]
                for g in gathers:
                    g.wait()

                pltpu.async_copy(
                    rowsv, out_hbm.at[pl.ds(r * _IDX_MINOR, _CHUNK)], osem)

                @pl.when(p < _PAIRS - 1)
                def _():
                    pltpu.async_copy(
                        idx_hbm.at[pl.ds(r + 2 * _K, _K)], idxv, isem)
            return carry

        lax.fori_loop(0, _PAIRS, step, 0)

        rlast = row0 + (_NCHUNKS - 2) * _K
        pltpu.make_async_copy(
            rows0, out_hbm.at[pl.ds(rlast * _IDX_MINOR, _CHUNK)], osem0).wait()
        pltpu.make_async_copy(
            rows1, out_hbm.at[pl.ds((rlast + _K) * _IDX_MINOR, _CHUNK)],
            osem1).wait()

    return body(idx2d, weights)


def kernel(token_ids, weights):
    idx2d = token_ids.astype(jnp.int32).reshape(_N // _IDX_MINOR, _IDX_MINOR)
    out = _sc_gather(idx2d, weights)
    return out.reshape(BATCH, HIST, EMBEDDING_DIM)
